# baseline (speedup 1.0000x reference)
import functools

import jax
import jax.numpy as jnp
from jax import lax
from jax.experimental import pallas as pl
from jax.experimental.pallas import tpu as pltpu
from jax.experimental.pallas import tpu_sc as plsc

_N = 10000
_E = 160000
_DIN = 256
_HID = 128
_HEADS = 8
_DH = 16
_OUT = 4

_NS = 16
_B = 128
_NB_TOTAL = _E // _B
_BASE_NB = _NB_TOTAL // _NS
_EXTRA = _NB_TOTAL - _BASE_NB * _NS
_RPT = _N // _NS


def _pre_body(x_ref, w_ref, b_ref, wa_ref, xs0_ref, xs1_ref, a_ref):
    xp = jnp.dot(x_ref[...], w_ref[...], preferred_element_type=jnp.float32)
    xp = xp + b_ref[...]
    a = jnp.dot(xp, wa_ref[...], preferred_element_type=jnp.float32)
    xs0_ref[...] = jnp.concatenate([xp, a[:, 0:16]], axis=1)
    xs1_ref[...] = jnp.concatenate([xp, a[:, 32:48]], axis=1)
    a_ref[...] = a


def _pre(x, W, b2, Wa):
    RB = 2000
    return pl.pallas_call(
        _pre_body,
        grid=(_N // RB,),
        in_specs=[
            pl.BlockSpec((RB, _DIN), lambda i: (i, 0)),
            pl.BlockSpec((_DIN, _HID), lambda i: (0, 0)),
            pl.BlockSpec((1, _HID), lambda i: (0, 0)),
            pl.BlockSpec((_HID, 64), lambda i: (0, 0)),
        ],
        out_specs=[
            pl.BlockSpec((RB, _HID + 16), lambda i: (i, 0)),
            pl.BlockSpec((RB, _HID + 16), lambda i: (i, 0)),
            pl.BlockSpec((RB, 64), lambda i: (i, 0)),
        ],
        out_shape=[
            jax.ShapeDtypeStruct((_N, _HID + 16), jnp.float32),
            jax.ShapeDtypeStruct((_N, _HID + 16), jnp.float32),
            jax.ShapeDtypeStruct((_N, 64), jnp.float32),
        ],
    )(x, W, b2, Wa)


_mesh = plsc.VectorSubcoreMesh(core_axis_name="c", subcore_axis_name="s")

_GDN = lax.GatherDimensionNumbers(
    offset_dims=(), collapsed_slice_dims=(0,), start_index_map=(0,))


def _lane_bcast(v, h):
    idx = jnp.full((16, 1), h, jnp.int32)
    return lax.gather(v, idx, _GDN, (1,),
                      mode=lax.GatherScatterMode.PROMISE_IN_BOUNDS)


@functools.partial(
    pl.kernel,
    out_type=jax.ShapeDtypeStruct((2, _N, _HID), jnp.float32),
    mesh=_mesh,
    scratch_types=[
        pltpu.VMEM_SHARED((_N, 16), jnp.float32),
        pltpu.VMEM_SHARED((_N, _HID), jnp.float32),
        pltpu.VMEM((2, _B), jnp.int32),
        pltpu.VMEM((_B, 16), jnp.float32),
        pltpu.VMEM((_B, 16), jnp.float32),
        pltpu.VMEM((_B, _HID + 16), jnp.float32),
        pltpu.VMEM((_B, _HID), jnp.float32),
    ],
    compiler_params=pltpu.CompilerParams(use_tc_tiling_on_sc=False),
)
def _sc_kernel(xs0, xs1, as0, ad0, as1, ad1, eb0, eb1, out_hbm,
               denom_sh, out_sh, eb, gb, gd, xsb, msg):
    s = lax.axis_index("s")
    c = lax.axis_index("c")
    lanes = lax.iota(jnp.int32, 16)
    msk = lanes < _HEADS
    zf = jnp.zeros((16,), jnp.float32)

    def _zero_row(e, _):
        gd[e] = zf
        for h in range(_HID // 16):
            msg[e, pl.ds(h * 16, 16)] = zf
        return 0

    lax.fori_loop(0, _B, _zero_row, 0)

    r0 = s * _RPT
    for i in range(4):
        pltpu.sync_copy(msg, out_sh.at[pl.ds(r0 + i * _B, _B)])
        pltpu.sync_copy(gd, denom_sh.at[pl.ds(r0 + i * _B, _B)])
    pltpu.sync_copy(msg.at[pl.ds(0, _RPT - 4 * _B)],
                    out_sh.at[pl.ds(r0 + 4 * _B, _RPT - 4 * _B)])
    pltpu.sync_copy(gd.at[pl.ds(0, _RPT - 4 * _B)],
                    denom_sh.at[pl.ds(r0 + 4 * _B, _RPT - 4 * _B)])
    plsc.subcore_barrier()

    def _run_mp(xsr, asr, adr, ebr, slot):
        def step1(ab):
            pltpu.sync_copy(ebr.at[ab], eb)
            pltpu.sync_copy(asr.at[eb.at[0]], gd)
            pltpu.sync_copy(adr.at[eb.at[1]], gb)

            def ebody(e, _):
                av = gd[e] + gb[e]
                av = jnp.where(av > 0, av, 0.2 * av)
                ex = jnp.exp(av)
                gd[e] = jnp.where(msk, ex, 0.0)
                return 0

            lax.fori_loop(0, _B, ebody, 0)
            pltpu.sync_copy(gd, denom_sh.at[eb.at[1]], add=True)

        def step2(ab):
            pltpu.sync_copy(ebr.at[ab], eb)
            pltpu.sync_copy(xsr.at[eb.at[0]], xsb)
            pltpu.sync_copy(adr.at[eb.at[1]], gb)
            pltpu.sync_copy(denom_sh.at[eb.at[1]], gd)

            def ebody(e, _):
                av = xsb[e, pl.ds(_HID, 16)] + gb[e]
                av = jnp.where(av > 0, av, 0.2 * av)
                ex = jnp.exp(av)
                den = gd[e]
                coef = jnp.where(msk, ex / den, 0.0)
                for h in range(_HEADS):
                    cv = _lane_bcast(coef, h)
                    xv = xsb[e, pl.ds(h * _DH, _DH)]
                    msg[e, pl.ds(h * _DH, _DH)] = xv * cv
                return 0

            lax.fori_loop(0, _B, ebody, 0)
            pltpu.sync_copy(msg, out_sh.at[eb.at[1]], add=True)

        def body1(k, _):
            step1(s * _BASE_NB + k)
            return 0

        lax.fori_loop(0, _BASE_NB, body1, 0)

        @pl.when(s < _EXTRA)
        def _():
            step1(_NS * _BASE_NB + s)

        plsc.subcore_barrier()

        def body2(k, _):
            step2(s * _BASE_NB + k)
            return 0

        lax.fori_loop(0, _BASE_NB, body2, 0)

        @pl.when(s < _EXTRA)
        def _():
            step2(_NS * _BASE_NB + s)

        plsc.subcore_barrier()

        r0w = 8 * (s * 78 + jnp.minimum(s, 2))
        pltpu.sync_copy(out_sh.at[pl.ds(r0w, 624)],
                        out_hbm.at[slot, pl.ds(r0w, 624)])

        @pl.when(s < 2)
        def _():
            pltpu.sync_copy(out_sh.at[pl.ds(r0w + 624, 8)],
                            out_hbm.at[slot, pl.ds(r0w + 624, 8)])

    @pl.when(c == 0)
    def _():
        _run_mp(xs0, as0, ad0, eb0, 0)

    @pl.when(c == 1)
    def _():
        _run_mp(xs1, as1, ad1, eb1, 1)


def _score_body(o_ref, kw_ref, kb_ref, q_ref, s_ref):
    m = pl.program_id(0)
    j = pl.program_id(1)
    r = jnp.maximum(o_ref[0], 0.0)
    kk = jnp.tanh(
        jnp.dot(r, kw_ref[...], preferred_element_type=jnp.float32)
        + kb_ref[...])
    part = jnp.sum(kk * q_ref[...])

    @pl.when(j == 0)
    def _():
        s_ref[m, 0] = 0.0

    s_ref[m, 0] += part


def _scores(o, kW, kb2, q2):
    RB = 2000
    return pl.pallas_call(
        _score_body,
        grid=(2, _N // RB),
        in_specs=[
            pl.BlockSpec((1, RB, _HID), lambda m, j: (m, j, 0)),
            pl.BlockSpec((_HID, _HID), lambda m, j: (0, 0)),
            pl.BlockSpec((1, _HID), lambda m, j: (0, 0)),
            pl.BlockSpec((1, _HID), lambda m, j: (0, 0)),
        ],
        out_specs=pl.BlockSpec((2, 1), lambda m, j: (0, 0),
                               memory_space=pltpu.SMEM),
        out_shape=jax.ShapeDtypeStruct((2, 1), jnp.float32),
    )(o, kW, kb2, q2)


def _fin_body(o_ref, a_ref, lw_ref, lb_ref, out_ref):
    a0 = a_ref[0]
    a1 = a_ref[1]
    r0 = jnp.maximum(o_ref[0], 0.0)
    r1 = jnp.maximum(o_ref[1], 0.0)
    m = a0 * r0 + a1 * r1
    out_ref[...] = (
        jnp.dot(m, lw_ref[...], preferred_element_type=jnp.float32)
        + lb_ref[...])


def _final(o, attn, lWp, lbp):
    RB = 2000
    return pl.pallas_call(
        _fin_body,
        grid=(_N // RB,),
        in_specs=[
            pl.BlockSpec((2, RB, _HID), lambda j: (0, j, 0)),
            pl.BlockSpec(memory_space=pltpu.SMEM),
            pl.BlockSpec((_HID, _HID), lambda j: (0, 0)),
            pl.BlockSpec((1, _HID), lambda j: (0, 0)),
        ],
        out_specs=pl.BlockSpec((RB, _HID), lambda j: (j, 0)),
        out_shape=jax.ShapeDtypeStruct((_N, _HID), jnp.float32),
    )(o, attn, lWp, lbp)


def _mk_att_mat(att):
    eye = jnp.eye(_HEADS, dtype=jnp.float32)
    return (att[:, :, None] * eye[:, None, :]).reshape(_HID, _HEADS)


def kernel(x_movie, edge_index_mp0, edge_index_mp1, W_proj, b_proj,
           att_src0, att_dst0, att_src1, att_dst1,
           k_lin_W, k_lin_b, q, lin_W, lin_b):
    z8 = jnp.zeros((_HID, _HEADS), jnp.float32)
    Wa = jnp.concatenate(
        [_mk_att_mat(att_src0), z8, _mk_att_mat(att_dst0), z8,
         _mk_att_mat(att_src1), z8, _mk_att_mat(att_dst1), z8], axis=1)

    xs0, xs1, A = _pre(x_movie, W_proj, b_proj.reshape(1, _HID), Wa)
    as0 = A[:, 0:16]
    ad0 = A[:, 16:32]
    as1 = A[:, 32:48]
    ad1 = A[:, 48:64]
    eb0 = edge_index_mp0.reshape(2, _NB_TOTAL, _B).transpose(1, 0, 2)
    eb1 = edge_index_mp1.reshape(2, _NB_TOTAL, _B).transpose(1, 0, 2)

    o = _sc_kernel(xs0, xs1, as0, ad0, as1, ad1, eb0, eb1)

    scores = _scores(o, k_lin_W, k_lin_b.reshape(1, _HID),
                     q.reshape(1, _HID))
    attn = jax.nn.softmax(scores[:, 0] / _N)

    lWp = jnp.zeros((_HID, _HID), jnp.float32).at[:, :_OUT].set(lin_W)
    lbp = jnp.zeros((1, _HID), jnp.float32).at[0, :_OUT].set(lin_b)
    outp = _final(o, attn, lWp, lbp)
    return outp[:, :_OUT]


# speedup vs baseline: 1.0634x; 1.0634x over previous
import functools

import jax
import jax.numpy as jnp
from jax import lax
from jax.experimental import pallas as pl
from jax.experimental.pallas import tpu as pltpu
from jax.experimental.pallas import tpu_sc as plsc

_N = 10000
_E = 160000
_DIN = 256
_HID = 128
_HEADS = 8
_DH = 16
_OUT = 4

_NS = 16
_B = 128
_NB_TOTAL = _E // _B
_BASE_NB = _NB_TOTAL // _NS
_EXTRA = _NB_TOTAL - _BASE_NB * _NS
_RPT = _N // _NS


def _pre_body(x_ref, w_ref, b_ref, wa_ref, xp_ref, a_ref):
    xp = jnp.dot(x_ref[...], w_ref[...], preferred_element_type=jnp.float32)
    xp = xp + b_ref[...]
    xp_ref[...] = xp
    a_ref[...] = jnp.dot(xp, wa_ref[...], preferred_element_type=jnp.float32)


def _pre(x, W, b2, Wa):
    RB = 2000
    return pl.pallas_call(
        _pre_body,
        grid=(_N // RB,),
        in_specs=[
            pl.BlockSpec((RB, _DIN), lambda i: (i, 0)),
            pl.BlockSpec((_DIN, _HID), lambda i: (0, 0)),
            pl.BlockSpec((1, _HID), lambda i: (0, 0)),
            pl.BlockSpec((_HID, 64), lambda i: (0, 0)),
        ],
        out_specs=[
            pl.BlockSpec((RB, _HID), lambda i: (i, 0)),
            pl.BlockSpec((RB, 64), lambda i: (i, 0)),
        ],
        out_shape=[
            jax.ShapeDtypeStruct((_N, _HID), jnp.float32),
            jax.ShapeDtypeStruct((_N, 64), jnp.float32),
        ],
    )(x, W, b2, Wa)


_mesh = plsc.VectorSubcoreMesh(core_axis_name="c", subcore_axis_name="s")

_GDN = lax.GatherDimensionNumbers(
    offset_dims=(), collapsed_slice_dims=(0,), start_index_map=(0,))


def _lane_bcast(v, h):
    idx = jnp.full((16, 1), h, jnp.int32)
    return lax.gather(v, idx, _GDN, (1,),
                      mode=lax.GatherScatterMode.PROMISE_IN_BOUNDS)


@functools.partial(
    pl.kernel,
    out_type=jax.ShapeDtypeStruct((2, _N, _HID), jnp.float32),
    mesh=_mesh,
    scratch_types=[
        pltpu.VMEM_SHARED((_N, 16), jnp.float32),
        pltpu.VMEM_SHARED((_N, _HID), jnp.float32),
        pltpu.VMEM((2, _B), jnp.int32),
        pltpu.VMEM((_B, 16), jnp.float32),
        pltpu.VMEM((_B, 16), jnp.float32),
        pltpu.VMEM((_B, 16), jnp.float32),
        pltpu.VMEM((_B, _HID), jnp.float32),
        pltpu.VMEM((_B, _HID), jnp.float32),
    ],
    compiler_params=pltpu.CompilerParams(use_tc_tiling_on_sc=False),
)
def _sc_kernel(xp, as0, ad0, as1, ad1, eb0, eb1, out_hbm,
               denom_sh, out_sh, eb, gb, gd, ga, xsb, msg):
    s = lax.axis_index("s")
    c = lax.axis_index("c")
    lanes = lax.iota(jnp.int32, 16)
    msk = lanes < _HEADS
    zf = jnp.zeros((16,), jnp.float32)

    def _zero_row(e, _):
        gd[e] = zf
        for h in range(_HID // 16):
            msg[e, pl.ds(h * 16, 16)] = zf
        return 0

    lax.fori_loop(0, _B, _zero_row, 0)

    r0 = s * _RPT
    for i in range(4):
        pltpu.sync_copy(msg, out_sh.at[pl.ds(r0 + i * _B, _B)])
        pltpu.sync_copy(gd, denom_sh.at[pl.ds(r0 + i * _B, _B)])
    pltpu.sync_copy(msg.at[pl.ds(0, _RPT - 4 * _B)],
                    out_sh.at[pl.ds(r0 + 4 * _B, _RPT - 4 * _B)])
    pltpu.sync_copy(gd.at[pl.ds(0, _RPT - 4 * _B)],
                    denom_sh.at[pl.ds(r0 + 4 * _B, _RPT - 4 * _B)])
    plsc.subcore_barrier()

    def _run_mp(asr, adr, ebr, slot):
        def step1(ab):
            pltpu.sync_copy(ebr.at[ab], eb)
            pltpu.sync_copy(asr.at[eb.at[0]], ga)
            pltpu.sync_copy(adr.at[eb.at[1]], gb)

            def ebody(e, _):
                av = ga[e] + gb[e]
                av = jnp.where(av > 0, av, 0.2 * av)
                ex = jnp.exp(av)
                gd[e] = jnp.where(msk, ex, 0.0)
                return 0

            lax.fori_loop(0, _B, ebody, 0)
            pltpu.sync_copy(gd, denom_sh.at[eb.at[1]], add=True)

        def step2(ab):
            pltpu.sync_copy(ebr.at[ab], eb)
            pltpu.sync_copy(xp.at[eb.at[0]], xsb)
            pltpu.sync_copy(asr.at[eb.at[0]], ga)
            pltpu.sync_copy(adr.at[eb.at[1]], gb)
            pltpu.sync_copy(denom_sh.at[eb.at[1]], gd)

            def ebody(e, _):
                av = ga[e] + gb[e]
                av = jnp.where(av > 0, av, 0.2 * av)
                ex = jnp.exp(av)
                den = gd[e]
                coef = jnp.where(msk, ex / den, 0.0)
                for h in range(_HEADS):
                    cv = _lane_bcast(coef, h)
                    xv = xsb[e, pl.ds(h * _DH, _DH)]
                    msg[e, pl.ds(h * _DH, _DH)] = xv * cv
                return 0

            lax.fori_loop(0, _B, ebody, 0)
            pltpu.sync_copy(msg, out_sh.at[eb.at[1]], add=True)

        def body1(k, _):
            step1(s * _BASE_NB + k)
            return 0

        lax.fori_loop(0, _BASE_NB, body1, 0)

        @pl.when(s < _EXTRA)
        def _():
            step1(_NS * _BASE_NB + s)

        plsc.subcore_barrier()

        def body2(k, _):
            step2(s * _BASE_NB + k)
            return 0

        lax.fori_loop(0, _BASE_NB, body2, 0)

        @pl.when(s < _EXTRA)
        def _():
            step2(_NS * _BASE_NB + s)

        plsc.subcore_barrier()

        r0w = 8 * (s * 78 + jnp.minimum(s, 2))
        pltpu.sync_copy(out_sh.at[pl.ds(r0w, 624)],
                        out_hbm.at[slot, pl.ds(r0w, 624)])

        @pl.when(s < 2)
        def _():
            pltpu.sync_copy(out_sh.at[pl.ds(r0w + 624, 8)],
                            out_hbm.at[slot, pl.ds(r0w + 624, 8)])

    @pl.when(c == 0)
    def _():
        _run_mp(as0, ad0, eb0, 0)

    @pl.when(c == 1)
    def _():
        _run_mp(as1, ad1, eb1, 1)


def _score_body(o_ref, kw_ref, kb_ref, q_ref, s_ref):
    m = pl.program_id(0)
    j = pl.program_id(1)
    r = jnp.maximum(o_ref[0], 0.0)
    kk = jnp.tanh(
        jnp.dot(r, kw_ref[...], preferred_element_type=jnp.float32)
        + kb_ref[...])
    part = jnp.sum(kk * q_ref[...])

    @pl.when(j == 0)
    def _():
        s_ref[m, 0] = 0.0

    s_ref[m, 0] += part


def _scores(o, kW, kb2, q2):
    RB = 2000
    return pl.pallas_call(
        _score_body,
        grid=(2, _N // RB),
        in_specs=[
            pl.BlockSpec((1, RB, _HID), lambda m, j: (m, j, 0)),
            pl.BlockSpec((_HID, _HID), lambda m, j: (0, 0)),
            pl.BlockSpec((1, _HID), lambda m, j: (0, 0)),
            pl.BlockSpec((1, _HID), lambda m, j: (0, 0)),
        ],
        out_specs=pl.BlockSpec((2, 1), lambda m, j: (0, 0),
                               memory_space=pltpu.SMEM),
        out_shape=jax.ShapeDtypeStruct((2, 1), jnp.float32),
    )(o, kW, kb2, q2)


def _fin_body(o_ref, a_ref, lw_ref, lb_ref, out_ref):
    a0 = a_ref[0]
    a1 = a_ref[1]
    r0 = jnp.maximum(o_ref[0], 0.0)
    r1 = jnp.maximum(o_ref[1], 0.0)
    m = a0 * r0 + a1 * r1
    out_ref[...] = (
        jnp.dot(m, lw_ref[...], preferred_element_type=jnp.float32)
        + lb_ref[...])


def _final(o, attn, lWp, lbp):
    RB = 2000
    return pl.pallas_call(
        _fin_body,
        grid=(_N // RB,),
        in_specs=[
            pl.BlockSpec((2, RB, _HID), lambda j: (0, j, 0)),
            pl.BlockSpec(memory_space=pltpu.SMEM),
            pl.BlockSpec((_HID, _HID), lambda j: (0, 0)),
            pl.BlockSpec((1, _HID), lambda j: (0, 0)),
        ],
        out_specs=pl.BlockSpec((RB, _HID), lambda j: (j, 0)),
        out_shape=jax.ShapeDtypeStruct((_N, _HID), jnp.float32),
    )(o, attn, lWp, lbp)


def _mk_att_mat(att):
    eye = jnp.eye(_HEADS, dtype=jnp.float32)
    return (att[:, :, None] * eye[:, None, :]).reshape(_HID, _HEADS)


def kernel(x_movie, edge_index_mp0, edge_index_mp1, W_proj, b_proj,
           att_src0, att_dst0, att_src1, att_dst1,
           k_lin_W, k_lin_b, q, lin_W, lin_b):
    z8 = jnp.zeros((_HID, _HEADS), jnp.float32)
    Wa = jnp.concatenate(
        [_mk_att_mat(att_src0), z8, _mk_att_mat(att_dst0), z8,
         _mk_att_mat(att_src1), z8, _mk_att_mat(att_dst1), z8], axis=1)

    xp, A = _pre(x_movie, W_proj, b_proj.reshape(1, _HID), Wa)
    as0 = A[:, 0:16]
    ad0 = A[:, 16:32]
    as1 = A[:, 32:48]
    ad1 = A[:, 48:64]
    eb0 = edge_index_mp0.reshape(2, _NB_TOTAL, _B).transpose(1, 0, 2)
    eb1 = edge_index_mp1.reshape(2, _NB_TOTAL, _B).transpose(1, 0, 2)

    o = _sc_kernel(xp, as0, ad0, as1, ad1, eb0, eb1)

    scores = _scores(o, k_lin_W, k_lin_b.reshape(1, _HID),
                     q.reshape(1, _HID))
    attn = jax.nn.softmax(scores[:, 0] / _N)

    lWp = jnp.zeros((_HID, _HID), jnp.float32).at[:, :_OUT].set(lin_W)
    lbp = jnp.zeros((1, _HID), jnp.float32).at[0, :_OUT].set(lin_b)
    outp = _final(o, attn, lWp, lbp)
    return outp[:, :_OUT]


# speedup vs baseline: 1.3171x; 1.2386x over previous
import functools

import jax
import jax.numpy as jnp
from jax import lax
from jax.experimental import pallas as pl
from jax.experimental.pallas import tpu as pltpu
from jax.experimental.pallas import tpu_sc as plsc

_N = 10000
_E = 160000
_DIN = 256
_HID = 128
_HEADS = 8
_DH = 16
_OUT = 4

_NS = 16
_B = 128
_NB_TOTAL = _E // _B
_BASE_NB = _NB_TOTAL // _NS
_EXTRA = _NB_TOTAL - _BASE_NB * _NS
_RPT = _N // _NS


def _pre_body(x_ref, w_ref, b_ref, wa_ref, xp_ref, a_ref):
    xp = jnp.dot(x_ref[...], w_ref[...], preferred_element_type=jnp.float32)
    xp = xp + b_ref[...]
    xp_ref[...] = xp
    a_ref[...] = jnp.dot(xp, wa_ref[...], preferred_element_type=jnp.float32)


def _pre(x, W, b2, Wa):
    RB = 2000
    return pl.pallas_call(
        _pre_body,
        grid=(_N // RB,),
        in_specs=[
            pl.BlockSpec((RB, _DIN), lambda i: (i, 0)),
            pl.BlockSpec((_DIN, _HID), lambda i: (0, 0)),
            pl.BlockSpec((1, _HID), lambda i: (0, 0)),
            pl.BlockSpec((_HID, 64), lambda i: (0, 0)),
        ],
        out_specs=[
            pl.BlockSpec((RB, _HID), lambda i: (i, 0)),
            pl.BlockSpec((RB, 64), lambda i: (i, 0)),
        ],
        out_shape=[
            jax.ShapeDtypeStruct((_N, _HID), jnp.float32),
            jax.ShapeDtypeStruct((_N, 64), jnp.float32),
        ],
    )(x, W, b2, Wa)


_mesh = plsc.VectorSubcoreMesh(core_axis_name="c", subcore_axis_name="s")

_GDN = lax.GatherDimensionNumbers(
    offset_dims=(), collapsed_slice_dims=(0,), start_index_map=(0,))


def _lane_bcast(v, h):
    idx = jnp.full((16, 1), h, jnp.int32)
    return lax.gather(v, idx, _GDN, (1,),
                      mode=lax.GatherScatterMode.PROMISE_IN_BOUNDS)


@functools.partial(
    pl.kernel,
    out_type=jax.ShapeDtypeStruct((2, _N, _HID), jnp.float32),
    mesh=_mesh,
    scratch_types=[
        pltpu.VMEM_SHARED((_N, 16), jnp.float32),
        pltpu.VMEM_SHARED((_N, _HID), jnp.float32),
        pltpu.VMEM((_B,), jnp.int32),
        pltpu.VMEM((_B,), jnp.int32),
        pltpu.VMEM((_B, 16), jnp.float32),
        pltpu.VMEM((_B, 16), jnp.float32),
        pltpu.VMEM((_B, 16), jnp.float32),
        pltpu.VMEM((_B, _HID), jnp.float32),
    ],
    compiler_params=pltpu.CompilerParams(use_tc_tiling_on_sc=False),
)
def _sc_kernel(xp, as0, ad0, as1, ad1, src0, dst0, src1, dst1, out_hbm,
               denom_sh, out_sh, sidx, didx, ga, gb, gd, xg):
    s = lax.axis_index("s")
    c = lax.axis_index("c")
    lanes = lax.iota(jnp.int32, 16)
    msk = lanes < _HEADS
    zf = jnp.zeros((16,), jnp.float32)

    def _zero_row(e, _):
        gd[e] = zf
        for h in range(_HID // 16):
            xg[e, pl.ds(h * 16, 16)] = zf
        return 0

    lax.fori_loop(0, _B, _zero_row, 0)

    r0 = s * _RPT
    for i in range(4):
        pltpu.sync_copy(xg, out_sh.at[pl.ds(r0 + i * _B, _B)])
        pltpu.sync_copy(gd, denom_sh.at[pl.ds(r0 + i * _B, _B)])
    pltpu.sync_copy(xg.at[pl.ds(0, _RPT - 4 * _B)],
                    out_sh.at[pl.ds(r0 + 4 * _B, _RPT - 4 * _B)])
    pltpu.sync_copy(gd.at[pl.ds(0, _RPT - 4 * _B)],
                    denom_sh.at[pl.ds(r0 + 4 * _B, _RPT - 4 * _B)])
    plsc.subcore_barrier()

    def _run_mp(asr, adr, src, dst, slot):
        def step1(ab):
            base = ab * _B
            pltpu.sync_copy(src.at[pl.ds(base, _B)], sidx)
            pltpu.sync_copy(dst.at[pl.ds(base, _B)], didx)
            pltpu.sync_copy(asr.at[sidx], ga)
            pltpu.sync_copy(adr.at[didx], gb)

            def ebody(i, _):
                for q in range(2):
                    e = 2 * i + q
                    av = ga[e] + gb[e]
                    av = jnp.where(av > 0, av, 0.2 * av)
                    ex = jnp.exp(av)
                    gd[e] = jnp.where(msk, ex, 0.0)
                return 0

            lax.fori_loop(0, _B // 2, ebody, 0)
            pltpu.sync_copy(gd, denom_sh.at[didx], add=True)

        def step2(ab):
            base = ab * _B
            pltpu.sync_copy(src.at[pl.ds(base, _B)], sidx)
            pltpu.sync_copy(dst.at[pl.ds(base, _B)], didx)
            pltpu.sync_copy(asr.at[sidx], ga)
            pltpu.sync_copy(adr.at[didx], gb)
            pltpu.sync_copy(denom_sh.at[didx], gd)
            pltpu.sync_copy(xp.at[sidx], xg)

            def ebody(i, _):
                for q in range(2):
                    e = 2 * i + q
                    av = ga[e] + gb[e]
                    av = jnp.where(av > 0, av, 0.2 * av)
                    ex = jnp.exp(av)
                    den = gd[e]
                    coef = jnp.where(msk, ex / den, 0.0)
                    for h in range(_HEADS):
                        cv = _lane_bcast(coef, h)
                        xv = xg[e, pl.ds(h * _DH, _DH)]
                        xg[e, pl.ds(h * _DH, _DH)] = xv * cv
                return 0

            lax.fori_loop(0, _B // 2, ebody, 0)
            pltpu.sync_copy(xg, out_sh.at[didx], add=True)

        def body1(k, _):
            step1(s * _BASE_NB + k)
            return 0

        lax.fori_loop(0, _BASE_NB, body1, 0)

        @pl.when(s < _EXTRA)
        def _():
            step1(_NS * _BASE_NB + s)

        plsc.subcore_barrier()

        def body2(k, _):
            step2(s * _BASE_NB + k)
            return 0

        lax.fori_loop(0, _BASE_NB, body2, 0)

        @pl.when(s < _EXTRA)
        def _():
            step2(_NS * _BASE_NB + s)

        plsc.subcore_barrier()

        r0w = 8 * (s * 78 + jnp.minimum(s, 2))
        pltpu.sync_copy(out_sh.at[pl.ds(r0w, 624)],
                        out_hbm.at[slot, pl.ds(r0w, 624)])

        @pl.when(s < 2)
        def _():
            pltpu.sync_copy(out_sh.at[pl.ds(r0w + 624, 8)],
                            out_hbm.at[slot, pl.ds(r0w + 624, 8)])

    @pl.when(c == 0)
    def _():
        _run_mp(as0, ad0, src0, dst0, 0)

    @pl.when(c == 1)
    def _():
        _run_mp(as1, ad1, src1, dst1, 1)


def _score_body(o_ref, kw_ref, kb_ref, q_ref, s_ref):
    m = pl.program_id(0)
    j = pl.program_id(1)
    r = jnp.maximum(o_ref[0], 0.0)
    kk = jnp.tanh(
        jnp.dot(r, kw_ref[...], preferred_element_type=jnp.float32)
        + kb_ref[...])
    part = jnp.sum(kk * q_ref[...])

    @pl.when(j == 0)
    def _():
        s_ref[m, 0] = 0.0

    s_ref[m, 0] += part


def _scores(o, kW, kb2, q2):
    RB = 2000
    return pl.pallas_call(
        _score_body,
        grid=(2, _N // RB),
        in_specs=[
            pl.BlockSpec((1, RB, _HID), lambda m, j: (m, j, 0)),
            pl.BlockSpec((_HID, _HID), lambda m, j: (0, 0)),
            pl.BlockSpec((1, _HID), lambda m, j: (0, 0)),
            pl.BlockSpec((1, _HID), lambda m, j: (0, 0)),
        ],
        out_specs=pl.BlockSpec((2, 1), lambda m, j: (0, 0),
                               memory_space=pltpu.SMEM),
        out_shape=jax.ShapeDtypeStruct((2, 1), jnp.float32),
    )(o, kW, kb2, q2)


def _fin_body(o_ref, a_ref, lw_ref, lb_ref, out_ref):
    a0 = a_ref[0]
    a1 = a_ref[1]
    r0 = jnp.maximum(o_ref[0], 0.0)
    r1 = jnp.maximum(o_ref[1], 0.0)
    m = a0 * r0 + a1 * r1
    out_ref[...] = (
        jnp.dot(m, lw_ref[...], preferred_element_type=jnp.float32)
        + lb_ref[...])


def _final(o, attn, lWp, lbp):
    RB = 2000
    return pl.pallas_call(
        _fin_body,
        grid=(_N // RB,),
        in_specs=[
            pl.BlockSpec((2, RB, _HID), lambda j: (0, j, 0)),
            pl.BlockSpec(memory_space=pltpu.SMEM),
            pl.BlockSpec((_HID, _HID), lambda j: (0, 0)),
            pl.BlockSpec((1, _HID), lambda j: (0, 0)),
        ],
        out_specs=pl.BlockSpec((RB, _HID), lambda j: (j, 0)),
        out_shape=jax.ShapeDtypeStruct((_N, _HID), jnp.float32),
    )(o, attn, lWp, lbp)


def _mk_att_mat(att):
    eye = jnp.eye(_HEADS, dtype=jnp.float32)
    return (att[:, :, None] * eye[:, None, :]).reshape(_HID, _HEADS)


def kernel(x_movie, edge_index_mp0, edge_index_mp1, W_proj, b_proj,
           att_src0, att_dst0, att_src1, att_dst1,
           k_lin_W, k_lin_b, q, lin_W, lin_b):
    z8 = jnp.zeros((_HID, _HEADS), jnp.float32)
    Wa = jnp.concatenate(
        [_mk_att_mat(att_src0), z8, _mk_att_mat(att_dst0), z8,
         _mk_att_mat(att_src1), z8, _mk_att_mat(att_dst1), z8], axis=1)

    xp, A = _pre(x_movie, W_proj, b_proj.reshape(1, _HID), Wa)
    as0 = A[:, 0:16]
    ad0 = A[:, 16:32]
    as1 = A[:, 32:48]
    ad1 = A[:, 48:64]
    o = _sc_kernel(xp, as0, ad0, as1, ad1,
                   edge_index_mp0[0], edge_index_mp0[1],
                   edge_index_mp1[0], edge_index_mp1[1])

    scores = _scores(o, k_lin_W, k_lin_b.reshape(1, _HID),
                     q.reshape(1, _HID))
    attn = jax.nn.softmax(scores[:, 0] / _N)

    lWp = jnp.zeros((_HID, _HID), jnp.float32).at[:, :_OUT].set(lin_W)
    lbp = jnp.zeros((1, _HID), jnp.float32).at[0, :_OUT].set(lin_b)
    outp = _final(o, attn, lWp, lbp)
    return outp[:, :_OUT]
